# Initial kernel scaffold; baseline (speedup 1.0000x reference)
import jax
import jax.numpy as jnp
from jax.experimental import pallas as pl


def kernel(X, edge_index, edge_weight, ggc_weight, gru_w_ih, gru_w_hh, gru_b_ih, gru_b_hh, lstm_w_ih, lstm_w_hh, lstm_b_ih, lstm_b_hh):
    raise NotImplementedError("write your pallas kernel here")


# speedup vs baseline: 2.6562x; 2.6562x over previous
import functools

import jax
import jax.numpy as jnp
from jax import lax
from jax.experimental import pallas as pl
from jax.experimental.pallas import tpu as pltpu
from jax.experimental.pallas import tpu_sc as plsc

_N = 10000
_E = 320000
_C = 128
_LH = 128
_NCORE = 2
_NSUB = 16
_NW = _NCORE * _NSUB
_CHUNK = 128
_NCH = 80
_EPT = _NCH * _CHUNK
_EPAD = _NW * _EPT
_NPAD = 10240
_RPT = _NPAD // _NSUB

_RB = 2000


def _sc_segment_sum(m, src_t, dst_t, ew_t):
  mesh = plsc.VectorSubcoreMesh(core_axis_name="c", subcore_axis_name="s")

  @functools.partial(
      pl.kernel,
      out_type=jax.ShapeDtypeStruct((_NCORE, _NPAD, _C), jnp.float32),
      mesh=mesh,
      scratch_types=[
          pltpu.VMEM((_NCH, _CHUNK), jnp.int32),
          pltpu.VMEM((_NCH, _CHUNK), jnp.int32),
          pltpu.VMEM((_NCH, _CHUNK), jnp.float32),
          pltpu.VMEM((_CHUNK, _C), jnp.float32),
          pltpu.VMEM_SHARED((_NPAD, _C), jnp.float32),
          pltpu.SemaphoreType.DMA,
      ],
  )
  def seg(m_hbm, src_hbm, dst_hbm, ew_hbm, out_hbm,
          src_v, dst_v, ew_v, rows_v, acc, sem):
    c = lax.axis_index("c")
    s = lax.axis_index("s")
    wid = c * _NSUB + s

    @pl.loop(0, _CHUNK)
    def _fill_zero(r):
      for v in range(_C // 16):
        rows_v[r, pl.ds(v * 16, 16)] = jnp.zeros((16,), jnp.float32)

    @pl.loop(0, _RPT // _CHUNK)
    def _zero_acc(b):
      pltpu.sync_copy(rows_v, acc.at[pl.ds(s * _RPT + b * _CHUNK, _CHUNK)])

    pltpu.sync_copy(src_hbm.at[wid], src_v)
    pltpu.sync_copy(dst_hbm.at[wid], dst_v)
    pltpu.sync_copy(ew_hbm.at[wid], ew_v)
    plsc.subcore_barrier()

    @pl.loop(0, _NCH)
    def _edges(j):
      pltpu.async_copy(m_hbm.at[src_v.at[j]], rows_v, sem).wait()

      @pl.loop(0, _CHUNK, step=16)
      def _scale(e0):
        wv = ew_v[j, pl.ds(e0, 16)]
        for k in range(16):
          w = wv[k]
          for v in range(_C // 16):
            sl = (e0 + k, pl.ds(v * 16, 16))
            rows_v[sl] = rows_v[sl] * w

      pltpu.sync_copy(rows_v, acc.at[dst_v.at[j]], add=True)

    plsc.subcore_barrier()
    pltpu.sync_copy(acc.at[pl.ds(s * _RPT, _RPT)],
                    out_hbm.at[c, pl.ds(s * _RPT, _RPT)])

  return seg(m, src_t, dst_t, ew_t)


def _tc_matmul(x, w):
  def body(x_ref, w_ref, o_ref):
    o_ref[...] = lax.dot_general(
        x_ref[...], w_ref[...], (((1,), (0,)), ((), ())),
        preferred_element_type=jnp.float32)

  return pl.pallas_call(
      body,
      grid=(_N // _RB,),
      in_specs=[pl.BlockSpec((_RB, _C), lambda i: (i, 0)),
                pl.BlockSpec((_C, _C), lambda i: (0, 0))],
      out_specs=pl.BlockSpec((_RB, _C), lambda i: (i, 0)),
      out_shape=jax.ShapeDtypeStruct((_N, _C), jnp.float32),
  )(x, w)


def _gru_block(p_ref, h, wih_ref, whh_ref, bih_ref, bhh_ref):
  agg = p_ref[0] + p_ref[1]
  gi = lax.dot_general(agg, wih_ref[...], (((1,), (1,)), ((), ())),
                       preferred_element_type=jnp.float32) + bih_ref[...]
  gh = lax.dot_general(h, whh_ref[...], (((1,), (1,)), ((), ())),
                       preferred_element_type=jnp.float32) + bhh_ref[...]
  r = jax.nn.sigmoid(gi[:, :_C] + gh[:, :_C])
  z = jax.nn.sigmoid(gi[:, _C:2 * _C] + gh[:, _C:2 * _C])
  n = jnp.tanh(gi[:, 2 * _C:] + r * gh[:, 2 * _C:])
  return (1.0 - z) * n + z * h


def _tc_gru_next(p, h, w_ih, w_hh, b_ih2, b_hh2, w_next):
  def body(p_ref, h_ref, wih_ref, whh_ref, bih_ref, bhh_ref, wn_ref,
           x_ref, m_ref):
    x = _gru_block(p_ref, h_ref[...], wih_ref, whh_ref, bih_ref, bhh_ref)
    x_ref[...] = x
    m_ref[...] = lax.dot_general(x, wn_ref[...], (((1,), (0,)), ((), ())),
                                 preferred_element_type=jnp.float32)

  return pl.pallas_call(
      body,
      grid=(_N // _RB,),
      in_specs=[
          pl.BlockSpec((_NCORE, _RB, _C), lambda i: (0, i, 0)),
          pl.BlockSpec((_RB, _C), lambda i: (i, 0)),
          pl.BlockSpec((3 * _C, _C), lambda i: (0, 0)),
          pl.BlockSpec((3 * _C, _C), lambda i: (0, 0)),
          pl.BlockSpec((1, 3 * _C), lambda i: (0, 0)),
          pl.BlockSpec((1, 3 * _C), lambda i: (0, 0)),
          pl.BlockSpec((_C, _C), lambda i: (0, 0)),
      ],
      out_specs=[pl.BlockSpec((_RB, _C), lambda i: (i, 0)),
                 pl.BlockSpec((_RB, _C), lambda i: (i, 0))],
      out_shape=[jax.ShapeDtypeStruct((_N, _C), jnp.float32),
                 jax.ShapeDtypeStruct((_N, _C), jnp.float32)],
  )(p, h, w_ih, w_hh, b_ih2, b_hh2, w_next)


def _tc_gru_lstm(p, h, w_ih, w_hh, b_ih2, b_hh2, lstm_w_ih, lb2):
  def body(p_ref, h_ref, wih_ref, whh_ref, bih_ref, bhh_ref, wl_ref, lb_ref,
           ht_ref, hn_ref, cn_ref):
    x = _gru_block(p_ref, h_ref[...], wih_ref, whh_ref, bih_ref, bhh_ref)
    gates = lax.dot_general(x, wl_ref[...], (((1,), (1,)), ((), ())),
                            preferred_element_type=jnp.float32) + lb_ref[...]
    ii = jax.nn.sigmoid(gates[:, :_LH])
    gg = jnp.tanh(gates[:, 2 * _LH:3 * _LH])
    oo = jax.nn.sigmoid(gates[:, 3 * _LH:])
    cn = ii * gg
    ht_ref[...] = x
    hn_ref[...] = oo * jnp.tanh(cn)
    cn_ref[...] = cn

  return pl.pallas_call(
      body,
      grid=(_N // _RB,),
      in_specs=[
          pl.BlockSpec((_NCORE, _RB, _C), lambda i: (0, i, 0)),
          pl.BlockSpec((_RB, _C), lambda i: (i, 0)),
          pl.BlockSpec((3 * _C, _C), lambda i: (0, 0)),
          pl.BlockSpec((3 * _C, _C), lambda i: (0, 0)),
          pl.BlockSpec((1, 3 * _C), lambda i: (0, 0)),
          pl.BlockSpec((1, 3 * _C), lambda i: (0, 0)),
          pl.BlockSpec((4 * _LH, _C), lambda i: (0, 0)),
          pl.BlockSpec((1, 4 * _LH), lambda i: (0, 0)),
      ],
      out_specs=[pl.BlockSpec((_RB, _C), lambda i: (i, 0)),
                 pl.BlockSpec((_RB, _LH), lambda i: (i, 0)),
                 pl.BlockSpec((_RB, _LH), lambda i: (i, 0))],
      out_shape=[jax.ShapeDtypeStruct((_N, _C), jnp.float32),
                 jax.ShapeDtypeStruct((_N, _LH), jnp.float32),
                 jax.ShapeDtypeStruct((_N, _LH), jnp.float32)],
  )(p, h, w_ih, w_hh, b_ih2, b_hh2, lstm_w_ih, lb2)


def kernel(X, edge_index, edge_weight, ggc_weight, gru_w_ih, gru_w_hh,
           gru_b_ih, gru_b_hh, lstm_w_ih, lstm_w_hh, lstm_b_ih, lstm_b_hh):
  pad = _EPAD - _E
  src_t = jnp.concatenate(
      [edge_index[0], jnp.zeros((pad,), jnp.int32)]).reshape(_NW, _NCH, _CHUNK)
  dst_t = jnp.concatenate(
      [edge_index[1], jnp.zeros((pad,), jnp.int32)]).reshape(_NW, _NCH, _CHUNK)
  ew_t = jnp.concatenate(
      [edge_weight, jnp.zeros((pad,), jnp.float32)]).reshape(_NW, _NCH, _CHUNK)
  bih2 = gru_b_ih.reshape(1, 3 * _C)
  bhh2 = gru_b_hh.reshape(1, 3 * _C)
  lb2 = (lstm_b_ih + lstm_b_hh).reshape(1, 4 * _LH)

  x = X
  m = _tc_matmul(x, ggc_weight[0])
  for i in range(3):
    p = _sc_segment_sum(m, src_t, dst_t, ew_t)
    if i < 2:
      x, m = _tc_gru_next(p, x, gru_w_ih, gru_w_hh, bih2, bhh2,
                          ggc_weight[i + 1])
    else:
      h_tilde, h_new, c_new = _tc_gru_lstm(p, x, gru_w_ih, gru_w_hh, bih2,
                                           bhh2, lstm_w_ih, lb2)
  return (h_tilde, h_new, c_new)


# speedup vs baseline: 3.4481x; 1.2981x over previous
import dataclasses
import functools

import jax
import jax.numpy as jnp
from jax import lax
from jax.experimental import pallas as pl
from jax.experimental.pallas import tpu as pltpu
from jax.experimental.pallas import tpu_sc as plsc

_N = 10000
_E = 320000
_C = 128
_LH = 128
_NCORE = 2
_NSUB = 16
_NW = _NCORE * _NSUB
_CHUNK = 128
_NCH = 80
_EPT = _NCH * _CHUNK
_EPAD = _NW * _EPT
_NPAD = 10240
_RPT = _NPAD // _NSUB

_RB = 2000


def _sc_segment_sum(m, comb_t):
  mesh = plsc.VectorSubcoreMesh(core_axis_name="c", subcore_axis_name="s")
  cp = pltpu.CompilerParams()
  if "needs_layout_passes" in pltpu.CompilerParams.__dataclass_fields__:
    cp = dataclasses.replace(cp, needs_layout_passes=False)

  @functools.partial(
      pl.kernel,
      out_type=jax.ShapeDtypeStruct((_NCORE, _NPAD, _C), jnp.float32),
      mesh=mesh,
      compiler_params=cp,
      scratch_types=[
          pltpu.VMEM((3, _CHUNK), jnp.int32),
          pltpu.VMEM((3, _CHUNK), jnp.int32),
          pltpu.VMEM((_CHUNK, _C), jnp.float32),
          pltpu.VMEM((_CHUNK, _C), jnp.float32),
          pltpu.VMEM_SHARED((_NPAD, _C), jnp.float32),
          pltpu.SemaphoreType.DMA,
          pltpu.SemaphoreType.DMA,
      ],
  )
  def seg(m_hbm, comb_hbm, out_hbm,
          idx0, idx1, rows0, rows1, acc, sem0, sem1):
    c = lax.axis_index("c")
    s = lax.axis_index("s")
    wid = c * _NSUB + s

    @pl.loop(0, _CHUNK)
    def _fill_zero(r):
      for v in range(_C // 16):
        rows0[r, pl.ds(v * 16, 16)] = jnp.zeros((16,), jnp.float32)

    @pl.loop(0, _RPT // _CHUNK)
    def _zero_acc(b):
      pltpu.sync_copy(rows0, acc.at[pl.ds(s * _RPT + b * _CHUNK, _CHUNK)])

    plsc.subcore_barrier()

    def scale_rows(rows_v, idx_v):
      @pl.loop(0, _CHUNK, step=16)
      def _scale(e0):
        wv = plsc.bitcast(idx_v[2, pl.ds(e0, 16)], jnp.float32)
        for k in range(16):
          w = wv[k]
          for v in range(_C // 16):
            sl = (e0 + k, pl.ds(v * 16, 16))
            rows_v[sl] = rows_v[sl] * w

    pltpu.sync_copy(comb_hbm.at[wid, 0], idx0)
    pltpu.async_copy(m_hbm.at[idx0.at[0]], rows0, sem0)
    pltpu.sync_copy(comb_hbm.at[wid, 1], idx1)
    pltpu.async_copy(m_hbm.at[idx1.at[0]], rows1, sem1)

    @pl.loop(0, _NCH, step=2)
    def _edges(j):
      pltpu.make_async_copy(m_hbm.at[idx0.at[0]], rows0, sem0).wait()
      scale_rows(rows0, idx0)
      pltpu.sync_copy(rows0, acc.at[idx0.at[1]], add=True)

      @pl.when(j + 2 < _NCH)
      def _next0():
        pltpu.sync_copy(comb_hbm.at[wid, j + 2], idx0)
        pltpu.async_copy(m_hbm.at[idx0.at[0]], rows0, sem0)

      pltpu.make_async_copy(m_hbm.at[idx1.at[0]], rows1, sem1).wait()
      scale_rows(rows1, idx1)
      pltpu.sync_copy(rows1, acc.at[idx1.at[1]], add=True)

      @pl.when(j + 3 < _NCH)
      def _next1():
        pltpu.sync_copy(comb_hbm.at[wid, j + 3], idx1)
        pltpu.async_copy(m_hbm.at[idx1.at[0]], rows1, sem1)

    plsc.subcore_barrier()
    pltpu.sync_copy(acc.at[pl.ds(s * _RPT, _RPT)],
                    out_hbm.at[c, pl.ds(s * _RPT, _RPT)])

  return seg(m, comb_t)


def _tc_matmul(x, w):
  def body(x_ref, w_ref, o_ref):
    o_ref[...] = lax.dot_general(
        x_ref[...], w_ref[...], (((1,), (0,)), ((), ())),
        preferred_element_type=jnp.float32)

  return pl.pallas_call(
      body,
      grid=(_N // _RB,),
      in_specs=[pl.BlockSpec((_RB, _C), lambda i: (i, 0)),
                pl.BlockSpec((_C, _C), lambda i: (0, 0))],
      out_specs=pl.BlockSpec((_RB, _C), lambda i: (i, 0)),
      out_shape=jax.ShapeDtypeStruct((_N, _C), jnp.float32),
  )(x, w)


def _gru_block(p_ref, h, wih_ref, whh_ref, bih_ref, bhh_ref):
  agg = p_ref[0] + p_ref[1]
  gi = lax.dot_general(agg, wih_ref[...], (((1,), (1,)), ((), ())),
                       preferred_element_type=jnp.float32) + bih_ref[...]
  gh = lax.dot_general(h, whh_ref[...], (((1,), (1,)), ((), ())),
                       preferred_element_type=jnp.float32) + bhh_ref[...]
  r = jax.nn.sigmoid(gi[:, :_C] + gh[:, :_C])
  z = jax.nn.sigmoid(gi[:, _C:2 * _C] + gh[:, _C:2 * _C])
  n = jnp.tanh(gi[:, 2 * _C:] + r * gh[:, 2 * _C:])
  return (1.0 - z) * n + z * h


def _tc_gru_next(p, h, w_ih, w_hh, b_ih2, b_hh2, w_next):
  def body(p_ref, h_ref, wih_ref, whh_ref, bih_ref, bhh_ref, wn_ref,
           x_ref, m_ref):
    x = _gru_block(p_ref, h_ref[...], wih_ref, whh_ref, bih_ref, bhh_ref)
    x_ref[...] = x
    m_ref[...] = lax.dot_general(x, wn_ref[...], (((1,), (0,)), ((), ())),
                                 preferred_element_type=jnp.float32)

  return pl.pallas_call(
      body,
      grid=(_N // _RB,),
      in_specs=[
          pl.BlockSpec((_NCORE, _RB, _C), lambda i: (0, i, 0)),
          pl.BlockSpec((_RB, _C), lambda i: (i, 0)),
          pl.BlockSpec((3 * _C, _C), lambda i: (0, 0)),
          pl.BlockSpec((3 * _C, _C), lambda i: (0, 0)),
          pl.BlockSpec((1, 3 * _C), lambda i: (0, 0)),
          pl.BlockSpec((1, 3 * _C), lambda i: (0, 0)),
          pl.BlockSpec((_C, _C), lambda i: (0, 0)),
      ],
      out_specs=[pl.BlockSpec((_RB, _C), lambda i: (i, 0)),
                 pl.BlockSpec((_RB, _C), lambda i: (i, 0))],
      out_shape=[jax.ShapeDtypeStruct((_N, _C), jnp.float32),
                 jax.ShapeDtypeStruct((_N, _C), jnp.float32)],
  )(p, h, w_ih, w_hh, b_ih2, b_hh2, w_next)


def _tc_gru_lstm(p, h, w_ih, w_hh, b_ih2, b_hh2, lstm_w_ih, lb2):
  def body(p_ref, h_ref, wih_ref, whh_ref, bih_ref, bhh_ref, wl_ref, lb_ref,
           ht_ref, hn_ref, cn_ref):
    x = _gru_block(p_ref, h_ref[...], wih_ref, whh_ref, bih_ref, bhh_ref)
    gates = lax.dot_general(x, wl_ref[...], (((1,), (1,)), ((), ())),
                            preferred_element_type=jnp.float32) + lb_ref[...]
    ii = jax.nn.sigmoid(gates[:, :_LH])
    gg = jnp.tanh(gates[:, 2 * _LH:3 * _LH])
    oo = jax.nn.sigmoid(gates[:, 3 * _LH:])
    cn = ii * gg
    ht_ref[...] = x
    hn_ref[...] = oo * jnp.tanh(cn)
    cn_ref[...] = cn

  return pl.pallas_call(
      body,
      grid=(_N // _RB,),
      in_specs=[
          pl.BlockSpec((_NCORE, _RB, _C), lambda i: (0, i, 0)),
          pl.BlockSpec((_RB, _C), lambda i: (i, 0)),
          pl.BlockSpec((3 * _C, _C), lambda i: (0, 0)),
          pl.BlockSpec((3 * _C, _C), lambda i: (0, 0)),
          pl.BlockSpec((1, 3 * _C), lambda i: (0, 0)),
          pl.BlockSpec((1, 3 * _C), lambda i: (0, 0)),
          pl.BlockSpec((4 * _LH, _C), lambda i: (0, 0)),
          pl.BlockSpec((1, 4 * _LH), lambda i: (0, 0)),
      ],
      out_specs=[pl.BlockSpec((_RB, _C), lambda i: (i, 0)),
                 pl.BlockSpec((_RB, _LH), lambda i: (i, 0)),
                 pl.BlockSpec((_RB, _LH), lambda i: (i, 0))],
      out_shape=[jax.ShapeDtypeStruct((_N, _C), jnp.float32),
                 jax.ShapeDtypeStruct((_N, _LH), jnp.float32),
                 jax.ShapeDtypeStruct((_N, _LH), jnp.float32)],
  )(p, h, w_ih, w_hh, b_ih2, b_hh2, lstm_w_ih, lb2)


def kernel(X, edge_index, edge_weight, ggc_weight, gru_w_ih, gru_w_hh,
           gru_b_ih, gru_b_hh, lstm_w_ih, lstm_w_hh, lstm_b_ih, lstm_b_hh):
  pad = _EPAD - _E
  src_t = jnp.concatenate(
      [edge_index[0], jnp.zeros((pad,), jnp.int32)]).reshape(_NW, _NCH, _CHUNK)
  dst_t = jnp.concatenate(
      [edge_index[1], jnp.zeros((pad,), jnp.int32)]).reshape(_NW, _NCH, _CHUNK)
  ew_t = lax.bitcast_convert_type(
      jnp.concatenate([edge_weight, jnp.zeros((pad,), jnp.float32)]),
      jnp.int32).reshape(_NW, _NCH, _CHUNK)
  comb_t = jnp.stack([src_t, dst_t, ew_t], axis=2)
  bih2 = gru_b_ih.reshape(1, 3 * _C)
  bhh2 = gru_b_hh.reshape(1, 3 * _C)
  lb2 = (lstm_b_ih + lstm_b_hh).reshape(1, 4 * _LH)

  x = X
  m = _tc_matmul(x, ggc_weight[0])
  for i in range(3):
    p = _sc_segment_sum(m, comb_t)
    if i < 2:
      x, m = _tc_gru_next(p, x, gru_w_ih, gru_w_hh, bih2, bhh2,
                          ggc_weight[i + 1])
    else:
      h_tilde, h_new, c_new = _tc_gru_lstm(p, x, gru_w_ih, gru_w_hh, bih2,
                                           bhh2, lstm_w_ih, lb2)
  return (h_tilde, h_new, c_new)


# speedup vs baseline: 3.5844x; 1.0395x over previous
import dataclasses
import functools

import jax
import jax.numpy as jnp
from jax import lax
from jax.experimental import pallas as pl
from jax.experimental.pallas import tpu as pltpu
from jax.experimental.pallas import tpu_sc as plsc

_N = 10000
_E = 320000
_C = 128
_LH = 128
_NCORE = 2
_NSUB = 16
_NW = _NCORE * _NSUB
_CHUNK = 128
_NCH0 = 126
_NCH1 = 34
_NCHT = _NSUB * (_NCH0 + _NCH1)
_EPAD = _NCHT * _CHUNK
_NPAD = 10240
_RPT = _NPAD // _NSUB

_RB = 2000


def _sc_segment_sum(m, comb_t):
  mesh = plsc.VectorSubcoreMesh(core_axis_name="c", subcore_axis_name="s")
  cp = pltpu.CompilerParams()
  if "needs_layout_passes" in pltpu.CompilerParams.__dataclass_fields__:
    cp = dataclasses.replace(cp, needs_layout_passes=False)

  @functools.partial(
      pl.kernel,
      out_type=jax.ShapeDtypeStruct((_NCORE, _NPAD, _C), jnp.float32),
      mesh=mesh,
      compiler_params=cp,
      scratch_types=[
          pltpu.VMEM((3, _CHUNK), jnp.int32),
          pltpu.VMEM((3, _CHUNK), jnp.int32),
          pltpu.VMEM((_CHUNK, _C), jnp.float32),
          pltpu.VMEM((_CHUNK, _C), jnp.float32),
          pltpu.VMEM_SHARED((_NPAD, _C), jnp.float32),
          pltpu.SemaphoreType.DMA,
          pltpu.SemaphoreType.DMA,
      ],
  )
  def seg(m_hbm, comb_hbm, out_hbm,
          idx0, idx1, rows0, rows1, acc, sem0, sem1):
    c = lax.axis_index("c")
    s = lax.axis_index("s")
    nch = jnp.where(c == 0, _NCH0, _NCH1)
    base = jnp.where(c == 0, s * _NCH0, _NSUB * _NCH0 + s * _NCH1)

    @pl.loop(0, _CHUNK)
    def _fill_zero(r):
      for v in range(_C // 16):
        rows0[r, pl.ds(v * 16, 16)] = jnp.zeros((16,), jnp.float32)

    @pl.loop(0, _RPT // _CHUNK)
    def _zero_acc(b):
      pltpu.sync_copy(rows0, acc.at[pl.ds(s * _RPT + b * _CHUNK, _CHUNK)])

    plsc.subcore_barrier()

    def scale_rows(rows_v, idx_v):
      @pl.loop(0, _CHUNK, step=16)
      def _scale(e0):
        wv = plsc.bitcast(idx_v[2, pl.ds(e0, 16)], jnp.float32)
        for k in range(16):
          w = wv[k]
          for v in range(_C // 16):
            sl = (e0 + k, pl.ds(v * 16, 16))
            rows_v[sl] = rows_v[sl] * w

    pltpu.sync_copy(comb_hbm.at[base], idx0)
    pltpu.async_copy(m_hbm.at[idx0.at[0]], rows0, sem0)
    pltpu.sync_copy(comb_hbm.at[base + 1], idx1)
    pltpu.async_copy(m_hbm.at[idx1.at[0]], rows1, sem1)

    @pl.loop(0, _NCH0, step=2)
    def _edges(j):
      @pl.when(j < nch)
      def _do0():
        pltpu.make_async_copy(m_hbm.at[idx0.at[0]], rows0, sem0).wait()
        scale_rows(rows0, idx0)
        pltpu.sync_copy(rows0, acc.at[idx0.at[1]], add=True)

        @pl.when(j + 2 < nch)
        def _next0():
          pltpu.sync_copy(comb_hbm.at[base + j + 2], idx0)
          pltpu.async_copy(m_hbm.at[idx0.at[0]], rows0, sem0)

      @pl.when(j + 1 < nch)
      def _do1():
        pltpu.make_async_copy(m_hbm.at[idx1.at[0]], rows1, sem1).wait()
        scale_rows(rows1, idx1)
        pltpu.sync_copy(rows1, acc.at[idx1.at[1]], add=True)

        @pl.when(j + 3 < nch)
        def _next1():
          pltpu.sync_copy(comb_hbm.at[base + j + 3], idx1)
          pltpu.async_copy(m_hbm.at[idx1.at[0]], rows1, sem1)

    plsc.subcore_barrier()
    pltpu.sync_copy(acc.at[pl.ds(s * _RPT, _RPT)],
                    out_hbm.at[c, pl.ds(s * _RPT, _RPT)])

  return seg(m, comb_t)


def _tc_matmul(x, w):
  def body(x_ref, w_ref, o_ref):
    o_ref[...] = lax.dot_general(
        x_ref[...], w_ref[...], (((1,), (0,)), ((), ())),
        preferred_element_type=jnp.float32)

  return pl.pallas_call(
      body,
      grid=(_N // _RB,),
      in_specs=[pl.BlockSpec((_RB, _C), lambda i: (i, 0)),
                pl.BlockSpec((_C, _C), lambda i: (0, 0))],
      out_specs=pl.BlockSpec((_RB, _C), lambda i: (i, 0)),
      out_shape=jax.ShapeDtypeStruct((_N, _C), jnp.float32),
  )(x, w)


def _gru_block(p_ref, h, wih_ref, whh_ref, bih_ref, bhh_ref):
  agg = p_ref[0] + p_ref[1]
  gi = lax.dot_general(agg, wih_ref[...], (((1,), (1,)), ((), ())),
                       preferred_element_type=jnp.float32) + bih_ref[...]
  gh = lax.dot_general(h, whh_ref[...], (((1,), (1,)), ((), ())),
                       preferred_element_type=jnp.float32) + bhh_ref[...]
  r = jax.nn.sigmoid(gi[:, :_C] + gh[:, :_C])
  z = jax.nn.sigmoid(gi[:, _C:2 * _C] + gh[:, _C:2 * _C])
  n = jnp.tanh(gi[:, 2 * _C:] + r * gh[:, 2 * _C:])
  return (1.0 - z) * n + z * h


def _tc_gru_next(p, h, w_ih, w_hh, b_ih2, b_hh2, w_next):
  def body(p_ref, h_ref, wih_ref, whh_ref, bih_ref, bhh_ref, wn_ref,
           x_ref, m_ref):
    x = _gru_block(p_ref, h_ref[...], wih_ref, whh_ref, bih_ref, bhh_ref)
    x_ref[...] = x
    m_ref[...] = lax.dot_general(x, wn_ref[...], (((1,), (0,)), ((), ())),
                                 preferred_element_type=jnp.float32)

  return pl.pallas_call(
      body,
      grid=(_N // _RB,),
      in_specs=[
          pl.BlockSpec((_NCORE, _RB, _C), lambda i: (0, i, 0)),
          pl.BlockSpec((_RB, _C), lambda i: (i, 0)),
          pl.BlockSpec((3 * _C, _C), lambda i: (0, 0)),
          pl.BlockSpec((3 * _C, _C), lambda i: (0, 0)),
          pl.BlockSpec((1, 3 * _C), lambda i: (0, 0)),
          pl.BlockSpec((1, 3 * _C), lambda i: (0, 0)),
          pl.BlockSpec((_C, _C), lambda i: (0, 0)),
      ],
      out_specs=[pl.BlockSpec((_RB, _C), lambda i: (i, 0)),
                 pl.BlockSpec((_RB, _C), lambda i: (i, 0))],
      out_shape=[jax.ShapeDtypeStruct((_N, _C), jnp.float32),
                 jax.ShapeDtypeStruct((_N, _C), jnp.float32)],
  )(p, h, w_ih, w_hh, b_ih2, b_hh2, w_next)


def _tc_gru_lstm(p, h, w_ih, w_hh, b_ih2, b_hh2, lstm_w_ih, lb2):
  def body(p_ref, h_ref, wih_ref, whh_ref, bih_ref, bhh_ref, wl_ref, lb_ref,
           ht_ref, hn_ref, cn_ref):
    x = _gru_block(p_ref, h_ref[...], wih_ref, whh_ref, bih_ref, bhh_ref)
    gates = lax.dot_general(x, wl_ref[...], (((1,), (1,)), ((), ())),
                            preferred_element_type=jnp.float32) + lb_ref[...]
    ii = jax.nn.sigmoid(gates[:, :_LH])
    gg = jnp.tanh(gates[:, 2 * _LH:3 * _LH])
    oo = jax.nn.sigmoid(gates[:, 3 * _LH:])
    cn = ii * gg
    ht_ref[...] = x
    hn_ref[...] = oo * jnp.tanh(cn)
    cn_ref[...] = cn

  return pl.pallas_call(
      body,
      grid=(_N // _RB,),
      in_specs=[
          pl.BlockSpec((_NCORE, _RB, _C), lambda i: (0, i, 0)),
          pl.BlockSpec((_RB, _C), lambda i: (i, 0)),
          pl.BlockSpec((3 * _C, _C), lambda i: (0, 0)),
          pl.BlockSpec((3 * _C, _C), lambda i: (0, 0)),
          pl.BlockSpec((1, 3 * _C), lambda i: (0, 0)),
          pl.BlockSpec((1, 3 * _C), lambda i: (0, 0)),
          pl.BlockSpec((4 * _LH, _C), lambda i: (0, 0)),
          pl.BlockSpec((1, 4 * _LH), lambda i: (0, 0)),
      ],
      out_specs=[pl.BlockSpec((_RB, _C), lambda i: (i, 0)),
                 pl.BlockSpec((_RB, _LH), lambda i: (i, 0)),
                 pl.BlockSpec((_RB, _LH), lambda i: (i, 0))],
      out_shape=[jax.ShapeDtypeStruct((_N, _C), jnp.float32),
                 jax.ShapeDtypeStruct((_N, _LH), jnp.float32),
                 jax.ShapeDtypeStruct((_N, _LH), jnp.float32)],
  )(p, h, w_ih, w_hh, b_ih2, b_hh2, lstm_w_ih, lb2)


def kernel(X, edge_index, edge_weight, ggc_weight, gru_w_ih, gru_w_hh,
           gru_b_ih, gru_b_hh, lstm_w_ih, lstm_w_hh, lstm_b_ih, lstm_b_hh):
  pad = _EPAD - _E
  src_t = jnp.concatenate(
      [edge_index[0], jnp.zeros((pad,), jnp.int32)]).reshape(_NCHT, _CHUNK)
  dst_t = jnp.concatenate(
      [edge_index[1], jnp.zeros((pad,), jnp.int32)]).reshape(_NCHT, _CHUNK)
  ew_t = lax.bitcast_convert_type(
      jnp.concatenate([edge_weight, jnp.zeros((pad,), jnp.float32)]),
      jnp.int32).reshape(_NCHT, _CHUNK)
  comb_t = jnp.stack([src_t, dst_t, ew_t], axis=1)
  bih2 = gru_b_ih.reshape(1, 3 * _C)
  bhh2 = gru_b_hh.reshape(1, 3 * _C)
  lb2 = (lstm_b_ih + lstm_b_hh).reshape(1, 4 * _LH)

  x = X
  m = _tc_matmul(x, ggc_weight[0])
  for i in range(3):
    p = _sc_segment_sum(m, comb_t)
    if i < 2:
      x, m = _tc_gru_next(p, x, gru_w_ih, gru_w_hh, bih2, bhh2,
                          ggc_weight[i + 1])
    else:
      h_tilde, h_new, c_new = _tc_gru_lstm(p, x, gru_w_ih, gru_w_hh, bih2,
                                           bhh2, lstm_w_ih, lb2)
  return (h_tilde, h_new, c_new)


# speedup vs baseline: 7.5285x; 2.1003x over previous
import dataclasses
import functools

import jax
import jax.numpy as jnp
from jax import lax
from jax.experimental import pallas as pl
from jax.experimental.pallas import tpu as pltpu
from jax.experimental.pallas import tpu_sc as plsc

_N = 10000
_E = 320000
_C = 128
_LH = 128
_NCORE = 2
_NSUB = 16
_NW = _NCORE * _NSUB
_CHUNK = 112
_NCH0 = 138
_NCH1 = 42
_NCHT = _NSUB * (_NCH0 + _NCH1)
_EPAD = _NCHT * _CHUNK
_NPAD = 10112
_RPT = _NPAD // _NSUB
_ZTAIL = _RPT - 5 * _CHUNK

_RB = 2000


def _sc_segment_sum(m, comb_t):
  mesh = plsc.VectorSubcoreMesh(core_axis_name="c", subcore_axis_name="s")
  cp = pltpu.CompilerParams()
  if "needs_layout_passes" in pltpu.CompilerParams.__dataclass_fields__:
    cp = dataclasses.replace(cp, needs_layout_passes=False)

  @functools.partial(
      pl.kernel,
      out_type=jax.ShapeDtypeStruct((_NCORE, _NPAD, _C), jnp.float32),
      mesh=mesh,
      compiler_params=cp,
      scratch_types=(
          [pltpu.VMEM((_CHUNK, _C), jnp.float32)] * 3
          + [pltpu.VMEM((3, _CHUNK), jnp.int32)] * 6
          + [pltpu.VMEM_SHARED((_NPAD, _C), jnp.float32)]
          + [pltpu.SemaphoreType.DMA] * 12
      ),
  )
  def seg(m_hbm, comb_hbm, out_hbm, *sc):
    rows = list(sc[0:3])
    idx = list(sc[3:9])
    acc = sc[9]
    gsem = list(sc[10:13])
    ssem = list(sc[13:16])
    isem = list(sc[16:22])

    c = lax.axis_index("c")
    s = lax.axis_index("s")
    nch = jnp.where(c == 0, _NCH0, _NCH1)
    base = jnp.where(c == 0, s * _NCH0, _NSUB * _NCH0 + s * _NCH1)

    @pl.loop(0, _CHUNK)
    def _fill_zero(r):
      for v in range(_C // 16):
        rows[0][r, pl.ds(v * 16, 16)] = jnp.zeros((16,), jnp.float32)

    @pl.loop(0, _RPT // _CHUNK)
    def _zero_acc(b):
      pltpu.sync_copy(rows[0], acc.at[pl.ds(s * _RPT + b * _CHUNK, _CHUNK)])

    pltpu.sync_copy(rows[0].at[pl.ds(0, _ZTAIL)],
                    acc.at[pl.ds(s * _RPT + 5 * _CHUNK, _ZTAIL)])

    plsc.subcore_barrier()

    def scale_rows(rows_v, idx_v):
      @pl.loop(0, _CHUNK, step=16)
      def _scale(e0):
        wv = plsc.bitcast(idx_v[2, pl.ds(e0, 16)], jnp.float32)
        for k in range(16):
          w = wv[k]
          for v in range(_C // 16):
            sl = (e0 + k, pl.ds(v * 16, 16))
            rows_v[sl] = rows_v[sl] * w

    for k in range(3):
      pltpu.sync_copy(comb_hbm.at[base + k], idx[k])
    for k in range(2):
      pltpu.async_copy(m_hbm.at[idx[k].at[0]], rows[k], gsem[k])

    @pl.loop(0, _NCH0, step=6)
    def _edges(j):
      for k_off in range(6):
        b = k_off % 3
        b2 = (k_off + 2) % 3
        i_cur = k_off
        i_2 = (k_off + 2) % 6
        i_3 = (k_off + 3) % 6
        k = j + k_off

        @pl.when(k < nch)
        def _blk(k=k, b=b, b2=b2, i_cur=i_cur, i_2=i_2, i_3=i_3,
                 k_off=k_off):
          pltpu.make_async_copy(m_hbm.at[idx[i_cur].at[0]], rows[b],
                                gsem[b]).wait()
          scale_rows(rows[b], idx[i_cur])
          pltpu.async_copy(rows[b], acc.at[idx[i_cur].at[1]], ssem[b],
                           add=True)

          @pl.when(k + 2 < nch)
          def _nxt():
            @pl.when(k >= 1)
            def _wait_idx():
              pltpu.make_async_copy(comb_hbm.at[base + k + 2], idx[i_2],
                                    isem[i_2]).wait()

            @pl.when(k >= 1)
            def _wait_scat():
              pltpu.make_async_copy(rows[b2], acc.at[idx[i_2].at[1]],
                                    ssem[b2]).wait()

            pltpu.async_copy(m_hbm.at[idx[i_2].at[0]], rows[b2], gsem[b2])

          @pl.when(k + 3 < nch)
          def _pref():
            pltpu.async_copy(comb_hbm.at[base + k + 3], idx[i_3], isem[i_3])

    for t in range(3):
      pltpu.make_async_copy(rows[t], acc.at[idx[t].at[1]], ssem[t]).wait()

    plsc.subcore_barrier()
    pltpu.sync_copy(acc.at[pl.ds(s * _RPT, _RPT)],
                    out_hbm.at[c, pl.ds(s * _RPT, _RPT)])

  return seg(m, comb_t)


def _tc_matmul(x, w):
  def body(x_ref, w_ref, o_ref):
    o_ref[...] = lax.dot_general(
        x_ref[...], w_ref[...], (((1,), (0,)), ((), ())),
        preferred_element_type=jnp.float32)

  return pl.pallas_call(
      body,
      grid=(_N // _RB,),
      in_specs=[pl.BlockSpec((_RB, _C), lambda i: (i, 0)),
                pl.BlockSpec((_C, _C), lambda i: (0, 0))],
      out_specs=pl.BlockSpec((_RB, _C), lambda i: (i, 0)),
      out_shape=jax.ShapeDtypeStruct((_N, _C), jnp.float32),
  )(x, w)


def _gru_block(p_ref, h, wih_ref, whh_ref, bih_ref, bhh_ref):
  agg = p_ref[0] + p_ref[1]
  gi = lax.dot_general(agg, wih_ref[...], (((1,), (1,)), ((), ())),
                       preferred_element_type=jnp.float32) + bih_ref[...]
  gh = lax.dot_general(h, whh_ref[...], (((1,), (1,)), ((), ())),
                       preferred_element_type=jnp.float32) + bhh_ref[...]
  r = jax.nn.sigmoid(gi[:, :_C] + gh[:, :_C])
  z = jax.nn.sigmoid(gi[:, _C:2 * _C] + gh[:, _C:2 * _C])
  n = jnp.tanh(gi[:, 2 * _C:] + r * gh[:, 2 * _C:])
  return (1.0 - z) * n + z * h


def _tc_gru_next(p, h, w_ih, w_hh, b_ih2, b_hh2, w_next):
  def body(p_ref, h_ref, wih_ref, whh_ref, bih_ref, bhh_ref, wn_ref,
           x_ref, m_ref):
    x = _gru_block(p_ref, h_ref[...], wih_ref, whh_ref, bih_ref, bhh_ref)
    x_ref[...] = x
    m_ref[...] = lax.dot_general(x, wn_ref[...], (((1,), (0,)), ((), ())),
                                 preferred_element_type=jnp.float32)

  return pl.pallas_call(
      body,
      grid=(_N // _RB,),
      in_specs=[
          pl.BlockSpec((_NCORE, _RB, _C), lambda i: (0, i, 0)),
          pl.BlockSpec((_RB, _C), lambda i: (i, 0)),
          pl.BlockSpec((3 * _C, _C), lambda i: (0, 0)),
          pl.BlockSpec((3 * _C, _C), lambda i: (0, 0)),
          pl.BlockSpec((1, 3 * _C), lambda i: (0, 0)),
          pl.BlockSpec((1, 3 * _C), lambda i: (0, 0)),
          pl.BlockSpec((_C, _C), lambda i: (0, 0)),
      ],
      out_specs=[pl.BlockSpec((_RB, _C), lambda i: (i, 0)),
                 pl.BlockSpec((_RB, _C), lambda i: (i, 0))],
      out_shape=[jax.ShapeDtypeStruct((_N, _C), jnp.float32),
                 jax.ShapeDtypeStruct((_N, _C), jnp.float32)],
  )(p, h, w_ih, w_hh, b_ih2, b_hh2, w_next)


def _tc_gru_lstm(p, h, w_ih, w_hh, b_ih2, b_hh2, lstm_w_ih, lb2):
  def body(p_ref, h_ref, wih_ref, whh_ref, bih_ref, bhh_ref, wl_ref, lb_ref,
           ht_ref, hn_ref, cn_ref):
    x = _gru_block(p_ref, h_ref[...], wih_ref, whh_ref, bih_ref, bhh_ref)
    gates = lax.dot_general(x, wl_ref[...], (((1,), (1,)), ((), ())),
                            preferred_element_type=jnp.float32) + lb_ref[...]
    ii = jax.nn.sigmoid(gates[:, :_LH])
    gg = jnp.tanh(gates[:, 2 * _LH:3 * _LH])
    oo = jax.nn.sigmoid(gates[:, 3 * _LH:])
    cn = ii * gg
    ht_ref[...] = x
    hn_ref[...] = oo * jnp.tanh(cn)
    cn_ref[...] = cn

  return pl.pallas_call(
      body,
      grid=(_N // _RB,),
      in_specs=[
          pl.BlockSpec((_NCORE, _RB, _C), lambda i: (0, i, 0)),
          pl.BlockSpec((_RB, _C), lambda i: (i, 0)),
          pl.BlockSpec((3 * _C, _C), lambda i: (0, 0)),
          pl.BlockSpec((3 * _C, _C), lambda i: (0, 0)),
          pl.BlockSpec((1, 3 * _C), lambda i: (0, 0)),
          pl.BlockSpec((1, 3 * _C), lambda i: (0, 0)),
          pl.BlockSpec((4 * _LH, _C), lambda i: (0, 0)),
          pl.BlockSpec((1, 4 * _LH), lambda i: (0, 0)),
      ],
      out_specs=[pl.BlockSpec((_RB, _C), lambda i: (i, 0)),
                 pl.BlockSpec((_RB, _LH), lambda i: (i, 0)),
                 pl.BlockSpec((_RB, _LH), lambda i: (i, 0))],
      out_shape=[jax.ShapeDtypeStruct((_N, _C), jnp.float32),
                 jax.ShapeDtypeStruct((_N, _LH), jnp.float32),
                 jax.ShapeDtypeStruct((_N, _LH), jnp.float32)],
  )(p, h, w_ih, w_hh, b_ih2, b_hh2, lstm_w_ih, lb2)


def kernel(X, edge_index, edge_weight, ggc_weight, gru_w_ih, gru_w_hh,
           gru_b_ih, gru_b_hh, lstm_w_ih, lstm_w_hh, lstm_b_ih, lstm_b_hh):
  pad = _EPAD - _E
  src_t = jnp.concatenate(
      [edge_index[0], jnp.zeros((pad,), jnp.int32)]).reshape(_NCHT, _CHUNK)
  dst_t = jnp.concatenate(
      [edge_index[1], jnp.zeros((pad,), jnp.int32)]).reshape(_NCHT, _CHUNK)
  ew_t = lax.bitcast_convert_type(
      jnp.concatenate([edge_weight, jnp.zeros((pad,), jnp.float32)]),
      jnp.int32).reshape(_NCHT, _CHUNK)
  comb_t = jnp.stack([src_t, dst_t, ew_t], axis=1)
  bih2 = gru_b_ih.reshape(1, 3 * _C)
  bhh2 = gru_b_hh.reshape(1, 3 * _C)
  lb2 = (lstm_b_ih + lstm_b_hh).reshape(1, 4 * _LH)

  x = X
  m = _tc_matmul(x, ggc_weight[0])
  for i in range(3):
    p = _sc_segment_sum(m, comb_t)
    if i < 2:
      x, m = _tc_gru_next(p, x, gru_w_ih, gru_w_hh, bih2, bhh2,
                          ggc_weight[i + 1])
    else:
      h_tilde, h_new, c_new = _tc_gru_lstm(p, x, gru_w_ih, gru_w_hh, bih2,
                                           bhh2, lstm_w_ih, lb2)
  return (h_tilde, h_new, c_new)
